# baseline (speedup 1.0000x reference)
import functools

import jax
import jax.numpy as jnp
from jax.experimental import pallas as pl
from jax.experimental.pallas import tpu as pltpu


def _vgl_kernel(feat_ref, adj_ref, wlp_ref, blp_ref, wenc_ref, benc_ref,
                wdec_ref, bdec_ref, out_ref, node_scr):
    s = pl.program_id(1)
    c = pl.program_id(2)
    n_s = pl.num_programs(1)
    n_c = pl.num_programs(2)

    feat = feat_ref[0, 0, 0].astype(jnp.bfloat16)
    adj = adj_ref[0, 0, 0].astype(jnp.bfloat16)
    w_lp = wlp_ref[0, 0].astype(jnp.bfloat16)
    b_lp = blp_ref[0, 0]

    t = jnp.dot(feat, w_lp, preferred_element_type=jnp.float32) + b_lp
    h = jnp.maximum(
        jnp.dot(adj, t.astype(jnp.bfloat16),
                preferred_element_type=jnp.float32), 0.0)
    contrib = h * (1.0 / n_c)

    n = feat.shape[0]
    row = pl.ds(s * n, n)

    @pl.when(c == 0)
    def _init():
        node_scr[row, :] = contrib

    @pl.when(c != 0)
    def _acc():
        node_scr[row, :] += contrib

    @pl.when(jnp.logical_and(s == n_s - 1, c == n_c - 1))
    def _finish():
        node = node_scr[:, :]
        m_tot = node.shape[0]
        node_c = node - jnp.mean(node, axis=1, keepdims=True)
        norm = jnp.sqrt(jnp.sum(node_c * node_c, axis=1, keepdims=True))
        node_n = (node_c / (norm + 1e-8)).astype(jnp.bfloat16)
        bg = jax.lax.dot_general(
            node_n, node_n, (((1,), (1,)), ((), ())),
            preferred_element_type=jnp.float32)
        bg = jnp.maximum(bg, 0.0)
        h3 = jnp.dot(bg.astype(jnp.bfloat16),
                     wenc_ref[:, :].astype(jnp.bfloat16),
                     preferred_element_type=jnp.float32)
        h3 = jnp.maximum(h3 + benc_ref[:, :], 0.0)
        w = jnp.sum(bg, axis=0, keepdims=True) * (1.0 / m_tot)
        pooled = jnp.dot(w, h3, preferred_element_type=jnp.float32)
        logits = jnp.dot(pooled, wdec_ref[:, :],
                         preferred_element_type=jnp.float32) + bdec_ref[:, :]
        b = pl.program_id(0)
        out_ref[pl.ds(b, 1), :] = jax.nn.sigmoid(logits)


@functools.partial(jax.jit, static_argnames=())
def kernel(feats, adjs, W_lp, b_lp, W_enc, b_enc, W_dec, b_dec):
    B, C, S, N, F = feats.shape
    d = W_lp.shape[-1]
    nc = W_dec.shape[-1]
    LANES = 128

    b_lp3 = b_lp.reshape(C, S, 1, d)
    b_enc2 = b_enc.reshape(1, d)
    W_dec_p = jnp.zeros((d, LANES), jnp.float32).at[:, :nc].set(W_dec)
    b_dec_p = jnp.zeros((1, LANES), jnp.float32).at[:, :nc].set(b_dec)

    grid = (B, S, C)
    out = pl.pallas_call(
        _vgl_kernel,
        grid=grid,
        in_specs=[
            pl.BlockSpec((1, 1, 1, N, F), lambda b, s, c: (b, c, s, 0, 0)),
            pl.BlockSpec((1, 1, 1, N, N), lambda b, s, c: (b, c, s, 0, 0)),
            pl.BlockSpec((1, 1, F, d), lambda b, s, c: (c, s, 0, 0)),
            pl.BlockSpec((1, 1, 1, d), lambda b, s, c: (c, s, 0, 0)),
            pl.BlockSpec((S * N, d), lambda b, s, c: (0, 0)),
            pl.BlockSpec((1, d), lambda b, s, c: (0, 0)),
            pl.BlockSpec((d, LANES), lambda b, s, c: (0, 0)),
            pl.BlockSpec((1, LANES), lambda b, s, c: (0, 0)),
        ],
        out_specs=pl.BlockSpec((B, LANES), lambda b, s, c: (0, 0)),
        out_shape=jax.ShapeDtypeStruct((B, LANES), jnp.float32),
        scratch_shapes=[pltpu.VMEM((S * N, d), jnp.float32)],
    )(feats, adjs, W_lp, b_lp3, W_enc, b_enc2, W_dec_p, b_dec_p)
    return out[:, :nc]


# speedup vs baseline: 2.2596x; 2.2596x over previous
import functools

import jax
import jax.numpy as jnp
from jax.experimental import pallas as pl
from jax.experimental.pallas import tpu as pltpu


def _vgl_kernel(feat_ref, adj_ref, wlp_ref, blp_ref, wenc_ref, benc_ref,
                wdec_ref, bdec_ref, out_ref):
    C = wlp_ref.shape[0]
    S = wlp_ref.shape[1]

    secs = []
    for j in range(S):
        acc = None
        for i in range(C):
            feat = feat_ref[0, i, j].astype(jnp.bfloat16)
            adj = adj_ref[0, i, j].astype(jnp.bfloat16)
            w_lp = wlp_ref[i, j].astype(jnp.bfloat16)
            b_lp = blp_ref[i, j]
            t = jnp.dot(feat, w_lp, preferred_element_type=jnp.float32) + b_lp
            h = jnp.maximum(
                jnp.dot(adj, t.astype(jnp.bfloat16),
                        preferred_element_type=jnp.float32), 0.0)
            acc = h if acc is None else acc + h
        secs.append(acc * (1.0 / C))
    node = jnp.concatenate(secs, axis=0)

    m_tot = node.shape[0]
    node_c = node - jnp.mean(node, axis=1, keepdims=True)
    norm = jnp.sqrt(jnp.sum(node_c * node_c, axis=1, keepdims=True))
    node_n = (node_c / (norm + 1e-8)).astype(jnp.bfloat16)
    bg = jax.lax.dot_general(
        node_n, node_n, (((1,), (1,)), ((), ())),
        preferred_element_type=jnp.float32)
    bg = jnp.maximum(bg, 0.0)
    h3 = jnp.dot(bg.astype(jnp.bfloat16),
                 wenc_ref[:, :].astype(jnp.bfloat16),
                 preferred_element_type=jnp.float32)
    h3 = jnp.maximum(h3 + benc_ref[:, :], 0.0)
    w = jnp.sum(bg, axis=0, keepdims=True) * (1.0 / m_tot)
    pooled = jnp.dot(w, h3, preferred_element_type=jnp.float32)
    logits = jnp.dot(pooled, wdec_ref[:, :],
                     preferred_element_type=jnp.float32) + bdec_ref[:, :]
    b = pl.program_id(0)
    out_ref[pl.ds(b, 1), :] = jax.nn.sigmoid(logits)


@functools.partial(jax.jit, static_argnames=())
def kernel(feats, adjs, W_lp, b_lp, W_enc, b_enc, W_dec, b_dec):
    B, C, S, N, F = feats.shape
    d = W_lp.shape[-1]
    nc = W_dec.shape[-1]
    LANES = 128

    b_lp3 = b_lp.reshape(C, S, 1, d)
    b_enc2 = b_enc.reshape(1, d)
    W_dec_p = jnp.zeros((d, LANES), jnp.float32).at[:, :nc].set(W_dec)
    b_dec_p = jnp.zeros((1, LANES), jnp.float32).at[:, :nc].set(b_dec)

    out = pl.pallas_call(
        _vgl_kernel,
        grid=(B,),
        in_specs=[
            pl.BlockSpec((1, C, S, N, F), lambda b: (b, 0, 0, 0, 0)),
            pl.BlockSpec((1, C, S, N, N), lambda b: (b, 0, 0, 0, 0)),
            pl.BlockSpec((C, S, F, d), lambda b: (0, 0, 0, 0)),
            pl.BlockSpec((C, S, 1, d), lambda b: (0, 0, 0, 0)),
            pl.BlockSpec((S * N, d), lambda b: (0, 0)),
            pl.BlockSpec((1, d), lambda b: (0, 0)),
            pl.BlockSpec((d, LANES), lambda b: (0, 0)),
            pl.BlockSpec((1, LANES), lambda b: (0, 0)),
        ],
        out_specs=pl.BlockSpec((B, LANES), lambda b: (0, 0)),
        out_shape=jax.ShapeDtypeStruct((B, LANES), jnp.float32),
    )(feats, adjs, W_lp, b_lp3, W_enc, b_enc2, W_dec_p, b_dec_p)
    return out[:, :nc]
